# Initial kernel scaffold; baseline (speedup 1.0000x reference)
import jax
import jax.numpy as jnp
from jax.experimental import pallas as pl


def kernel(x, codebook):
    raise NotImplementedError("write your pallas kernel here")


# speedup vs baseline: 1.0490x; 1.0490x over previous
import jax
import jax.numpy as jnp
from jax.experimental import pallas as pl

B, K, D = 8192, 8192, 64
BB = 512
KC = 2048


def _vq_kernel(x_ref, cb_ref, xsq_ref, csq_ref, out_ref):
    x = x_ref[...]
    x_sq = xsq_ref[...]

    def body(kc, carry):
        run_min, run_arg = carry
        c = cb_ref[pl.ds(kc * KC, KC), :]
        c_sq = csq_ref[:, pl.ds(kc * KC, KC)]
        cross = jax.lax.dot_general(
            x, c, (((1,), (1,)), ((), ())),
            preferred_element_type=jnp.float32)
        dists = x_sq + c_sq - 2.0 * cross
        loc_min = jnp.min(dists, axis=1, keepdims=True)
        idx = jax.lax.broadcasted_iota(jnp.int32, (BB, KC), 1) + kc * KC
        loc_arg = jnp.min(jnp.where(dists == loc_min, idx, K),
                          axis=1, keepdims=True)
        better = loc_min < run_min
        new_min = jnp.where(better, loc_min, run_min)
        new_min = new_min.astype(jnp.bfloat16).astype(jnp.float32)
        return (new_min, jnp.where(better, loc_arg, run_arg))

    init = (jnp.full((BB, 1), jnp.inf, jnp.float32),
            jnp.zeros((BB, 1), jnp.int32))
    _, arg = jax.lax.fori_loop(0, K // KC, body, init)
    out_ref[...] = arg


def kernel(x, codebook):
    x_sq = jnp.sum(x * x, axis=-1, keepdims=True)
    c_sq = jnp.sum(codebook * codebook, axis=-1)[None, :]
    out = pl.pallas_call(
        _vq_kernel,
        grid=(B // BB,),
        in_specs=[
            pl.BlockSpec((BB, D), lambda i: (i, 0)),
            pl.BlockSpec((K, D), lambda i: (0, 0)),
            pl.BlockSpec((BB, 1), lambda i: (i, 0)),
            pl.BlockSpec((1, K), lambda i: (0, 0)),
        ],
        out_specs=pl.BlockSpec((BB, 1), lambda i: (i, 0)),
        out_shape=jax.ShapeDtypeStruct((B, 1), jnp.int32),
    )(x, codebook, x_sq, c_sq)
    return out.reshape(B)
